# baseline (speedup 1.0000x reference)
import jax
import jax.numpy as jnp
from jax import lax
from jax.experimental import pallas as pl
from jax.experimental.pallas import tpu as pltpu
from jax.experimental.pallas import tpu_sc as plsc

_B = 8
_D = 768


def _scs_body(idx_hbm, table_hbm, out_hbm, idx_s, sem):
    pltpu.sync_copy(idx_hbm, idx_s)
    for i in range(_B):
        pltpu.async_copy(table_hbm.at[idx_s[i]], out_hbm.at[i], sem)
    pltpu.make_async_copy(table_hbm.at[pl.ds(0, _B)], out_hbm, sem).wait()


def kernel(x, table):
    idx = x.astype(jnp.int32)
    mesh = plsc.ScalarSubcoreMesh(axis_name="c", num_cores=1)
    k = pl.kernel(
        _scs_body,
        mesh=mesh,
        out_type=jax.ShapeDtypeStruct((_B, _D), jnp.float32),
        scratch_types=[
            pltpu.SMEM((_B,), jnp.int32),
            pltpu.SemaphoreType.DMA,
        ],
    )
    return k(idx, table)
